# baseline (speedup 1.0000x reference)
import functools

import jax
import jax.numpy as jnp
from jax import lax
from jax.experimental import pallas as pl
from jax.experimental.pallas import tpu as pltpu
from jax.experimental.pallas import tpu_sc as plsc

B, L, V, P, D = 4096, 200, 1000, 256, 128
N = B * L


def _pair_body(tok_ref, pos_ref, g_ref, b_ref, out_ref,
               ctg_ref, cpg_ref, rstd_ref):
    @pl.when(pl.program_id(0) == 0)
    def _():
        t = tok_ref[:]
        p = pos_ref[:]
        g = g_ref[0][None, :]
        mt = jnp.mean(t, axis=-1, keepdims=True)
        mp = jnp.mean(p, axis=-1, keepdims=True)
        ct = t - mt
        cp = p - mp
        var_t = jnp.mean(ct * ct, axis=-1, keepdims=True)
        var_p = jnp.mean(cp * cp, axis=-1, keepdims=True)
        cov = lax.dot_general(ct, cp, (((1,), (1,)), ((), ())),
                              preferred_element_type=jnp.float32)
        var = var_t + var_p.reshape(1, P) + (2.0 / D) * cov
        rstd_ref[:] = 1.0 / jnp.sqrt(var + 1e-5)
        ctg_ref[:] = ct * g
        cpg_ref[:] = cp * g

    i = pl.program_id(0)
    BV = out_ref.shape[0]
    ctg = ctg_ref[pl.ds(i * BV, BV), :]
    rstd = rstd_ref[pl.ds(i * BV, BV), :]
    out_ref[:] = (ctg[:, None, :] + cpg_ref[:][None, :, :]) \
        * rstd[:, :, None] + b_ref[0][None, None, :]


def _pair_table(token_table, pos_table, gamma, beta):
    BV = 8
    grid = (V // BV,)
    return pl.pallas_call(
        _pair_body,
        grid=grid,
        in_specs=[
            pl.BlockSpec((V, D), lambda i: (0, 0)),
            pl.BlockSpec((P, D), lambda i: (0, 0)),
            pl.BlockSpec((1, D), lambda i: (0, 0)),
            pl.BlockSpec((1, D), lambda i: (0, 0)),
        ],
        out_specs=pl.BlockSpec((BV, P, D), lambda i: (i, 0, 0)),
        out_shape=jax.ShapeDtypeStruct((V, P, D), jnp.float32),
        scratch_shapes=[
            pltpu.VMEM((V, D), jnp.float32),
            pltpu.VMEM((P, D), jnp.float32),
            pltpu.VMEM((V, P), jnp.float32),
        ],
    )(token_table, pos_table, gamma.reshape(1, D), beta.reshape(1, D))


def _make_sc_pair_idx():
    info = plsc.get_sparse_core_info()
    NC, NS = info.num_cores, info.num_subcores
    NW = NC * NS
    n_per_w = N // NW

    mesh = plsc.VectorSubcoreMesh(core_axis_name="c", subcore_axis_name="s")

    @functools.partial(
        pl.kernel,
        mesh=mesh,
        out_type=jax.ShapeDtypeStruct((N,), jnp.int32),
        scratch_types=[
            pltpu.VMEM((n_per_w,), jnp.int32),
            pltpu.VMEM((n_per_w,), jnp.int32),
        ],
    )
    def sc_pair_idx(tok_hbm, pos_hbm, idx_hbm, tok_v, pos_v):
        wid = lax.axis_index("s") * NC + lax.axis_index("c")
        w_base = wid * n_per_w
        pltpu.sync_copy(tok_hbm.at[pl.ds(w_base, n_per_w)], tok_v)
        pltpu.sync_copy(pos_hbm.at[pl.ds(w_base, n_per_w)], pos_v)

        def body(i, _):
            for j in range(4):
                s = pl.ds(i * 64 + j * 16, 16)
                tok_v[s] = tok_v[s] * P + pos_v[s]
            return ()

        lax.fori_loop(0, n_per_w // 64, body, (), unroll=False)
        pltpu.sync_copy(tok_v, idx_hbm.at[pl.ds(w_base, n_per_w)])

    return sc_pair_idx


def _make_sc_gather():
    info = plsc.get_sparse_core_info()
    NC, NS = info.num_cores, info.num_subcores
    NW = NC * NS
    n_per_w = N // NW
    G = 128
    NG = n_per_w // G
    NBUF = 5

    mesh = plsc.VectorSubcoreMesh(core_axis_name="c", subcore_axis_name="s")

    @functools.partial(
        pl.kernel,
        mesh=mesh,
        out_type=jax.ShapeDtypeStruct((N, D), jnp.float32),
        scratch_types=[
            pltpu.VMEM((n_per_w,), jnp.int32),
            [pltpu.VMEM((G, D), jnp.float32) for _ in range(NBUF)],
            [pltpu.SemaphoreType.DMA for _ in range(NBUF)],
            [pltpu.SemaphoreType.DMA for _ in range(NBUF)],
        ],
    )
    def sc_gather(table_hbm, idx_hbm, out_hbm, idx_v, rows, gsems, ssems):
        wid = lax.axis_index("s") * NC + lax.axis_index("c")
        w_base = wid * n_per_w

        pltpu.sync_copy(idx_hbm.at[pl.ds(w_base, n_per_w)], idx_v)

        def start_gather(g, nb):
            pltpu.async_copy(table_hbm.at[idx_v.at[pl.ds(g * G, G)]],
                             rows[nb], gsems[nb])

        def start_scatter(g, b):
            pltpu.async_copy(rows[b], out_hbm.at[pl.ds(w_base + g * G, G)],
                             ssems[b])

        def drain(sem, b):
            pltpu.make_async_copy(out_hbm.at[pl.ds(0, G)], rows[b], sem).wait()

        start_gather(0, 0)

        def outer(h, _):
            for b in range(NBUF):
                g = h * NBUF + b
                nb = (b + 1) % NBUF

                if b == NBUF - 1:
                    @pl.when(h < NG // NBUF - 1)
                    def _():
                        drain(ssems[nb], nb)
                        start_gather(g + 1, nb)
                else:
                    @pl.when(h > 0)
                    def _():
                        drain(ssems[nb], nb)
                    start_gather(g + 1, nb)

                drain(gsems[b], b)
                start_scatter(g, b)
            return ()

        lax.fori_loop(0, NG // NBUF, outer, (), unroll=False)

        for g in range(NG - NBUF, NG):
            drain(ssems[g % NBUF], g % NBUF)

    return sc_gather


def kernel(board_tokens, board_positions, token_table, pos_table, gamma, beta):
    tok = board_tokens.astype(jnp.int32).reshape(N)
    pos = board_positions.astype(jnp.int32).reshape(N)
    pair_idx = _make_sc_pair_idx()(tok, pos)
    table = _pair_table(token_table, pos_table, gamma, beta).reshape(V * P, D)
    out = _make_sc_gather()(table, pair_idx)
    return out.reshape(B, L, D)


# speedup vs baseline: 1.0904x; 1.0904x over previous
import functools

import jax
import jax.numpy as jnp
from jax import lax
from jax.experimental import pallas as pl
from jax.experimental.pallas import tpu as pltpu
from jax.experimental.pallas import tpu_sc as plsc

B, L, V, P, D = 4096, 200, 1000, 256, 128
N = B * L


def _pair_body(tok_ref, pos_ref, g_ref, b_ref, out_ref,
               ctg_ref, cpg_ref, rstd_ref):
    @pl.when(pl.program_id(0) == 0)
    def _():
        t = tok_ref[:]
        p = pos_ref[:]
        g = g_ref[0][None, :]
        mt = jnp.mean(t, axis=-1, keepdims=True)
        mp = jnp.mean(p, axis=-1, keepdims=True)
        ct = t - mt
        cp = p - mp
        var_t = jnp.mean(ct * ct, axis=-1, keepdims=True)
        var_p = jnp.mean(cp * cp, axis=-1, keepdims=True)
        cov = lax.dot_general(ct, cp, (((1,), (1,)), ((), ())),
                              preferred_element_type=jnp.float32)
        var = var_t + var_p.reshape(1, P) + (2.0 / D) * cov
        rstd_ref[:] = 1.0 / jnp.sqrt(var + 1e-5)
        ctg_ref[:] = ct * g
        cpg_ref[:] = cp * g

    i = pl.program_id(0)
    BV = out_ref.shape[0]
    ctg = ctg_ref[pl.ds(i * BV, BV), :]
    rstd = rstd_ref[pl.ds(i * BV, BV), :]
    out_ref[:] = (ctg[:, None, :] + cpg_ref[:][None, :, :]) \
        * rstd[:, :, None] + b_ref[0][None, None, :]


def _pair_table(token_table, pos_table, gamma, beta):
    BV = 40
    grid = (V // BV,)
    return pl.pallas_call(
        _pair_body,
        grid=grid,
        in_specs=[
            pl.BlockSpec((V, D), lambda i: (0, 0)),
            pl.BlockSpec((P, D), lambda i: (0, 0)),
            pl.BlockSpec((1, D), lambda i: (0, 0)),
            pl.BlockSpec((1, D), lambda i: (0, 0)),
        ],
        out_specs=pl.BlockSpec((BV, P, D), lambda i: (i, 0, 0)),
        out_shape=jax.ShapeDtypeStruct((V, P, D), jnp.float32),
        scratch_shapes=[
            pltpu.VMEM((V, D), jnp.float32),
            pltpu.VMEM((P, D), jnp.float32),
            pltpu.VMEM((V, P), jnp.float32),
        ],
    )(token_table, pos_table, gamma.reshape(1, D), beta.reshape(1, D))


def _make_sc_pair_idx():
    info = plsc.get_sparse_core_info()
    NC, NS = info.num_cores, info.num_subcores
    NW = NC * NS
    n_per_w = N // NW

    mesh = plsc.VectorSubcoreMesh(core_axis_name="c", subcore_axis_name="s")

    @functools.partial(
        pl.kernel,
        mesh=mesh,
        out_type=jax.ShapeDtypeStruct((N,), jnp.int32),
        scratch_types=[
            pltpu.VMEM((n_per_w,), jnp.int32),
            pltpu.VMEM((n_per_w,), jnp.int32),
        ],
    )
    def sc_pair_idx(tok_hbm, pos_hbm, idx_hbm, tok_v, pos_v):
        wid = lax.axis_index("s") * NC + lax.axis_index("c")
        w_base = wid * n_per_w
        pltpu.sync_copy(tok_hbm.at[pl.ds(w_base, n_per_w)], tok_v)
        pltpu.sync_copy(pos_hbm.at[pl.ds(w_base, n_per_w)], pos_v)

        def body(i, _):
            for j in range(4):
                s = pl.ds(i * 64 + j * 16, 16)
                tok_v[s] = tok_v[s] * P + pos_v[s]
            return ()

        lax.fori_loop(0, n_per_w // 64, body, (), unroll=False)
        pltpu.sync_copy(tok_v, idx_hbm.at[pl.ds(w_base, n_per_w)])

    return sc_pair_idx


def _make_sc_gather():
    info = plsc.get_sparse_core_info()
    NC, NS = info.num_cores, info.num_subcores
    NW = NC * NS
    n_per_w = N // NW
    G = 128
    NG = n_per_w // G
    NBUF = 5

    mesh = plsc.VectorSubcoreMesh(core_axis_name="c", subcore_axis_name="s")

    @functools.partial(
        pl.kernel,
        mesh=mesh,
        out_type=jax.ShapeDtypeStruct((N, D), jnp.float32),
        scratch_types=[
            pltpu.VMEM((n_per_w,), jnp.int32),
            [pltpu.VMEM((G, D), jnp.float32) for _ in range(NBUF)],
            [pltpu.SemaphoreType.DMA for _ in range(NBUF)],
            [pltpu.SemaphoreType.DMA for _ in range(NBUF)],
        ],
    )
    def sc_gather(table_hbm, idx_hbm, out_hbm, idx_v, rows, gsems, ssems):
        wid = lax.axis_index("s") * NC + lax.axis_index("c")
        w_base = wid * n_per_w

        pltpu.sync_copy(idx_hbm.at[pl.ds(w_base, n_per_w)], idx_v)

        def start_gather(g, nb):
            pltpu.async_copy(table_hbm.at[idx_v.at[pl.ds(g * G, G)]],
                             rows[nb], gsems[nb])

        def start_scatter(g, b):
            pltpu.async_copy(rows[b], out_hbm.at[pl.ds(w_base + g * G, G)],
                             ssems[b])

        def drain(sem, b):
            pltpu.make_async_copy(out_hbm.at[pl.ds(0, G)], rows[b], sem).wait()

        start_gather(0, 0)

        def outer(h, _):
            for b in range(NBUF):
                g = h * NBUF + b
                nb = (b + 1) % NBUF

                if b == NBUF - 1:
                    @pl.when(h < NG // NBUF - 1)
                    def _():
                        drain(ssems[nb], nb)
                        start_gather(g + 1, nb)
                else:
                    @pl.when(h > 0)
                    def _():
                        drain(ssems[nb], nb)
                    start_gather(g + 1, nb)

                drain(gsems[b], b)
                start_scatter(g, b)
            return ()

        lax.fori_loop(0, NG // NBUF, outer, (), unroll=False)

        for g in range(NG - NBUF, NG):
            drain(ssems[g % NBUF], g % NBUF)

    return sc_gather


def kernel(board_tokens, board_positions, token_table, pos_table, gamma, beta):
    tok = board_tokens.astype(jnp.int32).reshape(N)
    pos = board_positions.astype(jnp.int32).reshape(N)
    pair_idx = _make_sc_pair_idx()(tok, pos)
    table = _pair_table(token_table, pos_table, gamma, beta).reshape(V * P, D)
    out = _make_sc_gather()(table, pair_idx)
    return out.reshape(B, L, D)


# speedup vs baseline: 1.0942x; 1.0035x over previous
import functools

import jax
import jax.numpy as jnp
from jax import lax
from jax.experimental import pallas as pl
from jax.experimental.pallas import tpu as pltpu
from jax.experimental.pallas import tpu_sc as plsc

B, L, V, P, D = 4096, 200, 1000, 256, 128
N = B * L


def _pair_body(tok_ref, pos_ref, g_ref, b_ref, out_ref,
               ctg_ref, cpg_ref, rstd_ref):
    @pl.when(pl.program_id(0) == 0)
    def _():
        t = tok_ref[:]
        p = pos_ref[:]
        g = g_ref[0][None, :]
        mt = jnp.mean(t, axis=-1, keepdims=True)
        mp = jnp.mean(p, axis=-1, keepdims=True)
        ct = t - mt
        cp = p - mp
        var_t = jnp.mean(ct * ct, axis=-1, keepdims=True)
        var_p = jnp.mean(cp * cp, axis=-1, keepdims=True)
        cov = lax.dot_general(ct, cp, (((1,), (1,)), ((), ())),
                              preferred_element_type=jnp.float32)
        var = var_t + var_p.reshape(1, P) + (2.0 / D) * cov
        rstd_ref[:] = 1.0 / jnp.sqrt(var + 1e-5)
        ctg_ref[:] = ct * g
        cpg_ref[:] = cp * g

    i = pl.program_id(0)
    BV = out_ref.shape[0]
    ctg = ctg_ref[pl.ds(i * BV, BV), :]
    rstd = rstd_ref[pl.ds(i * BV, BV), :]
    out_ref[:] = (ctg[:, None, :] + cpg_ref[:][None, :, :]) \
        * rstd[:, :, None] + b_ref[0][None, None, :]


def _pair_table(token_table, pos_table, gamma, beta):
    BV = 200
    grid = (V // BV,)
    return pl.pallas_call(
        _pair_body,
        grid=grid,
        in_specs=[
            pl.BlockSpec((V, D), lambda i: (0, 0)),
            pl.BlockSpec((P, D), lambda i: (0, 0)),
            pl.BlockSpec((1, D), lambda i: (0, 0)),
            pl.BlockSpec((1, D), lambda i: (0, 0)),
        ],
        out_specs=pl.BlockSpec((BV, P, D), lambda i: (i, 0, 0)),
        out_shape=jax.ShapeDtypeStruct((V, P, D), jnp.float32),
        scratch_shapes=[
            pltpu.VMEM((V, D), jnp.float32),
            pltpu.VMEM((P, D), jnp.float32),
            pltpu.VMEM((V, P), jnp.float32),
        ],
    )(token_table, pos_table, gamma.reshape(1, D), beta.reshape(1, D))


def _make_sc_pair_idx():
    info = plsc.get_sparse_core_info()
    NC, NS = info.num_cores, info.num_subcores
    NW = NC * NS
    n_per_w = N // NW

    mesh = plsc.VectorSubcoreMesh(core_axis_name="c", subcore_axis_name="s")

    @functools.partial(
        pl.kernel,
        mesh=mesh,
        out_type=jax.ShapeDtypeStruct((N,), jnp.int32),
        scratch_types=[
            pltpu.VMEM((n_per_w,), jnp.int32),
            pltpu.VMEM((n_per_w,), jnp.int32),
        ],
    )
    def sc_pair_idx(tok_hbm, pos_hbm, idx_hbm, tok_v, pos_v):
        wid = lax.axis_index("s") * NC + lax.axis_index("c")
        w_base = wid * n_per_w
        pltpu.sync_copy(tok_hbm.at[pl.ds(w_base, n_per_w)], tok_v)
        pltpu.sync_copy(pos_hbm.at[pl.ds(w_base, n_per_w)], pos_v)

        def body(i, _):
            for j in range(4):
                s = pl.ds(i * 64 + j * 16, 16)
                tok_v[s] = tok_v[s] * P + pos_v[s]
            return ()

        lax.fori_loop(0, n_per_w // 64, body, (), unroll=False)
        pltpu.sync_copy(tok_v, idx_hbm.at[pl.ds(w_base, n_per_w)])

    return sc_pair_idx


def _make_sc_gather():
    info = plsc.get_sparse_core_info()
    NC, NS = info.num_cores, info.num_subcores
    NW = NC * NS
    n_per_w = N // NW
    G = 128
    NG = n_per_w // G
    NBUF = 5

    mesh = plsc.VectorSubcoreMesh(core_axis_name="c", subcore_axis_name="s")

    @functools.partial(
        pl.kernel,
        mesh=mesh,
        out_type=jax.ShapeDtypeStruct((N, D), jnp.float32),
        scratch_types=[
            pltpu.VMEM((n_per_w,), jnp.int32),
            [pltpu.VMEM((G, D), jnp.float32) for _ in range(NBUF)],
            [pltpu.SemaphoreType.DMA for _ in range(NBUF)],
            [pltpu.SemaphoreType.DMA for _ in range(NBUF)],
        ],
    )
    def sc_gather(table_hbm, idx_hbm, out_hbm, idx_v, rows, gsems, ssems):
        wid = lax.axis_index("s") * NC + lax.axis_index("c")
        w_base = wid * n_per_w

        pltpu.sync_copy(idx_hbm.at[pl.ds(w_base, n_per_w)], idx_v)

        def start_gather(g, nb):
            pltpu.async_copy(table_hbm.at[idx_v.at[pl.ds(g * G, G)]],
                             rows[nb], gsems[nb])

        def start_scatter(g, b):
            pltpu.async_copy(rows[b], out_hbm.at[pl.ds(w_base + g * G, G)],
                             ssems[b])

        def drain(sem, b):
            pltpu.make_async_copy(out_hbm.at[pl.ds(0, G)], rows[b], sem).wait()

        start_gather(0, 0)

        def outer(h, _):
            for b in range(NBUF):
                g = h * NBUF + b
                nb = (b + 1) % NBUF

                if b == NBUF - 1:
                    @pl.when(h < NG // NBUF - 1)
                    def _():
                        drain(ssems[nb], nb)
                        start_gather(g + 1, nb)
                else:
                    @pl.when(h > 0)
                    def _():
                        drain(ssems[nb], nb)
                    start_gather(g + 1, nb)

                drain(gsems[b], b)
                start_scatter(g, b)
            return ()

        lax.fori_loop(0, NG // NBUF, outer, (), unroll=False)

        for g in range(NG - NBUF, NG):
            drain(ssems[g % NBUF], g % NBUF)

    return sc_gather


def kernel(board_tokens, board_positions, token_table, pos_table, gamma, beta):
    tok = board_tokens.astype(jnp.int32).reshape(N)
    pos = board_positions.astype(jnp.int32).reshape(N)
    pair_idx = _make_sc_pair_idx()(tok, pos)
    table = _pair_table(token_table, pos_table, gamma, beta).reshape(V * P, D)
    out = _make_sc_gather()(table, pair_idx)
    return out.reshape(B, L, D)
